# Initial kernel scaffold; baseline (speedup 1.0000x reference)
import jax
import jax.numpy as jnp
from jax.experimental import pallas as pl


def kernel(x1, x2, W, b):
    raise NotImplementedError("write your pallas kernel here")


# speedup vs baseline: 1.8943x; 1.8943x over previous
import jax
import jax.numpy as jnp
from jax.experimental import pallas as pl


def _ew_kernel(x1_ref, x2_ref, w1_ref, b1_ref, w2_ref, b2_ref, y1_ref, y2_ref):
    y1_ref[...] = jnp.maximum(x1_ref[...], 0.0) * w1_ref[...] + b1_ref[...]
    y2_ref[...] = jnp.maximum(x2_ref[...], 0.0) * w2_ref[...] + b2_ref[...]


def kernel(x1, x2, W, b):
    n, d = x1.shape
    w1 = W[:d].reshape(1, d)
    w2 = W[d:].reshape(1, d)
    b1 = b[:d].reshape(1, d)
    b2 = b[d:].reshape(1, d)

    block_rows = 4000
    grid = (n // block_rows,)
    bs_x = pl.BlockSpec((block_rows, d), lambda i: (i, 0))
    bs_w = pl.BlockSpec((1, d), lambda i: (0, 0))

    y1, y2 = pl.pallas_call(
        _ew_kernel,
        grid=grid,
        in_specs=[bs_x, bs_x, bs_w, bs_w, bs_w, bs_w],
        out_specs=[bs_x, bs_x],
        out_shape=[jax.ShapeDtypeStruct((n, d), x1.dtype)] * 2,
    )(x1, x2, w1, b1, w2, b2)
    return (y1, y2)


# speedup vs baseline: 1.9449x; 1.0267x over previous
import jax
import jax.numpy as jnp
from jax.experimental import pallas as pl


def _ew_kernel(x1_ref, x2_ref, w1_ref, b1_ref, w2_ref, b2_ref, y1_ref, y2_ref):
    y1_ref[...] = jnp.maximum(x1_ref[...], 0.0) * w1_ref[...] + b1_ref[...]
    y2_ref[...] = jnp.maximum(x2_ref[...], 0.0) * w2_ref[...] + b2_ref[...]


def kernel(x1, x2, W, b):
    n, d = x1.shape
    w1 = W[:d].reshape(1, d)
    w2 = W[d:].reshape(1, d)
    b1 = b[:d].reshape(1, d)
    b2 = b[d:].reshape(1, d)

    block_rows = 10000
    grid = (n // block_rows,)
    bs_x = pl.BlockSpec((block_rows, d), lambda i: (i, 0))
    bs_w = pl.BlockSpec((1, d), lambda i: (0, 0))

    y1, y2 = pl.pallas_call(
        _ew_kernel,
        grid=grid,
        in_specs=[bs_x, bs_x, bs_w, bs_w, bs_w, bs_w],
        out_specs=[bs_x, bs_x],
        out_shape=[jax.ShapeDtypeStruct((n, d), x1.dtype)] * 2,
    )(x1, x2, w1, b1, w2, b2)
    return (y1, y2)


# speedup vs baseline: 1.9450x; 1.0000x over previous
import jax
import jax.numpy as jnp
from jax.experimental import pallas as pl
from jax.experimental.pallas import tpu as pltpu


def _ew_kernel(x1_ref, x2_ref, w1_ref, b1_ref, w2_ref, b2_ref, y1_ref, y2_ref):
    y1_ref[...] = jnp.maximum(x1_ref[...], 0.0) * w1_ref[...] + b1_ref[...]
    y2_ref[...] = jnp.maximum(x2_ref[...], 0.0) * w2_ref[...] + b2_ref[...]


def kernel(x1, x2, W, b):
    n, d = x1.shape
    w1 = W[:d].reshape(1, d)
    w2 = W[d:].reshape(1, d)
    b1 = b[:d].reshape(1, d)
    b2 = b[d:].reshape(1, d)

    block_rows = 10000
    grid = (n // block_rows,)
    bs_x = pl.BlockSpec((block_rows, d), lambda i: (i, 0))
    bs_w = pl.BlockSpec((1, d), lambda i: (0, 0))

    y1, y2 = pl.pallas_call(
        _ew_kernel,
        grid=grid,
        in_specs=[bs_x, bs_x, bs_w, bs_w, bs_w, bs_w],
        out_specs=[bs_x, bs_x],
        out_shape=[jax.ShapeDtypeStruct((n, d), x1.dtype)] * 2,
        compiler_params=pltpu.CompilerParams(
            dimension_semantics=("parallel",),
        ),
    )(x1, x2, w1, b1, w2, b2)
    return (y1, y2)
